# baseline (speedup 1.0000x reference)
import jax
import jax.numpy as jnp
from jax import lax
from jax.experimental import pallas as pl
from jax.experimental.pallas import tpu as pltpu
from jax.experimental.pallas import tpu_sc as plsc

N = 50000
E = 1000000
NSUB = 16
CHUNK = 128
K = 16
G = 31
T_SUB = K * G
R = 5
D = 4
EP = NSUB * CHUNK * T_SUB
EROWS = NSUB * T_SUB
ZROWS = 3128
ACC_ROWS = NSUB * ZROWS
OROWS = 3128
OLAST = N - 15 * OROWS
CW = 8
BLK = 2000

_SC_PARAMS = pltpu.CompilerParams(use_tc_tiling_on_sc=False)


def _mesh():
    return plsc.VectorSubcoreMesh(core_axis_name="c", subcore_axis_name="s")


def _make_agg():
    out_type = [jax.ShapeDtypeStruct((N, 32), jnp.float32),
                jax.ShapeDtypeStruct((N, 32), jnp.float32)]
    scratch = [
        pltpu.VMEM_SHARED((ACC_ROWS, 32), jnp.float32),
        pltpu.VMEM((2, K, CHUNK), jnp.int32),
        pltpu.VMEM((3, K, CHUNK), jnp.int32),
        pltpu.VMEM((R, CHUNK, 32), jnp.float32),
        pltpu.SemaphoreType.DMA,
        pltpu.SemaphoreType.DMA,
        pltpu.SemaphoreType.DMA,
    ]

    def body(h0, h1, esrc, edst, z32, agg0, agg1, acc, sidx, didx, rows,
             sem_i, sem_g, sem_s):
        c = lax.axis_index("c")
        s = lax.axis_index("s")

        pltpu.sync_copy(z32, acc.at[pl.ds(s * ZROWS, ZROWS)])
        plsc.subcore_barrier()

        B = s * T_SUB

        def idx_start(g):
            pltpu.async_copy(esrc.at[pl.ds(B + g * K, K)],
                             sidx.at[lax.rem(g, 2)], sem_i)
            pltpu.async_copy(edst.at[pl.ds(B + g * K, K)],
                             didx.at[lax.rem(g, 3)], sem_i)

        def idx_wait(g):
            pltpu.make_async_copy(esrc.at[pl.ds(B + g * K, K)],
                                  sidx.at[lax.rem(g, 2)], sem_i).wait()
            pltpu.make_async_copy(edst.at[pl.ds(B + g * K, K)],
                                  didx.at[lax.rem(g, 3)], sem_i).wait()

        def run(h):
            def gather_start(q):
                qg, qk = q // K, lax.rem(q, K)
                pltpu.async_copy(h.at[sidx.at[lax.rem(qg, 2), qk]],
                                 rows.at[lax.rem(q, R)], sem_g)

            def gather_wait(q):
                qg, qk = q // K, lax.rem(q, K)
                pltpu.make_async_copy(h.at[sidx.at[lax.rem(qg, 2), qk]],
                                      rows.at[lax.rem(q, R)], sem_g).wait()

            def scatter_wait(q):
                qg, qk = q // K, lax.rem(q, K)
                pltpu.make_async_copy(
                    rows.at[lax.rem(q, R)],
                    acc.at[didx.at[lax.rem(qg, 3), qk]], sem_s).wait()

            idx_start(0)
            idx_wait(0)
            idx_start(1)
            for q in range(D):
                gather_start(q)

            def step(j, carry):
                g, k = j // K, lax.rem(j, K)
                gather_wait(j)

                @pl.when((k == 0) & (g >= 1) & (g + 1 < G))
                def _():
                    idx_start(g + 1)

                pltpu.async_copy(rows.at[lax.rem(j, R)],
                                 acc.at[didx.at[lax.rem(g, 3), k]],
                                 sem_s, add=True)

                q = j + D

                @pl.when(q < T_SUB)
                def _():
                    @pl.when((lax.rem(q, K) == 0) & (q // K >= 1))
                    def _():
                        idx_wait(q // K)

                    @pl.when(q >= R)
                    def _():
                        scatter_wait(q - R)

                    gather_start(q)

                return carry

            lax.fori_loop(0, T_SUB, step, 0)
            for r in range(R):
                scatter_wait(T_SUB - R + r)

        @pl.when(c == 0)
        def _():
            run(h0)

        @pl.when(c == 1)
        def _():
            run(h1)

        plsc.subcore_barrier()

        lo = pl.multiple_of(s * OROWS, 8)
        agg = [agg0, agg1]
        for cc in (0, 1):
            @pl.when((c == cc) & (s < 15))
            def _(cc=cc):
                pltpu.sync_copy(acc.at[pl.ds(lo, OROWS)],
                                agg[cc].at[pl.ds(lo, OROWS)])

            @pl.when((c == cc) & (s == 15))
            def _(cc=cc):
                pltpu.sync_copy(acc.at[pl.ds(15 * OROWS, OLAST)],
                                agg[cc].at[pl.ds(15 * OROWS, OLAST)])

    return pl.kernel(body, out_type=out_type, mesh=_mesh(),
                     scratch_types=scratch, compiler_params=_SC_PARAMS)


_agg = _make_agg()


def _make_count():
    out_type = [jax.ShapeDtypeStruct((N, CW), jnp.float32)] * 4
    scratch = [
        pltpu.VMEM_SHARED((2, ACC_ROWS, CW), jnp.float32),
        pltpu.VMEM((3, K, CHUNK), jnp.int32),
        pltpu.VMEM((CHUNK, CW), jnp.float32),
        pltpu.SemaphoreType.DMA,
        pltpu.SemaphoreType.DMA,
    ]

    def body(ed_rp, ed_pr, ed_rc, ed_cr, zc, ones,
             o_rp, o_pr, o_rc, o_cr, cnt, didx, ones_v, sem_i, sem_c):
        c = lax.axis_index("c")
        s = lax.axis_index("s")

        pltpu.sync_copy(zc, cnt.at[0, pl.ds(s * ZROWS, ZROWS)])
        pltpu.sync_copy(zc, cnt.at[1, pl.ds(s * ZROWS, ZROWS)])
        pltpu.sync_copy(ones, ones_v)
        plsc.subcore_barrier()

        B = s * T_SUB

        def run(ed, slot):
            cnt_s = cnt.at[slot]

            def idx_start(g):
                pltpu.async_copy(ed.at[pl.ds(B + g * K, K)],
                                 didx.at[lax.rem(g, 3)], sem_i)

            def idx_wait(g):
                pltpu.make_async_copy(ed.at[pl.ds(B + g * K, K)],
                                      didx.at[lax.rem(g, 3)], sem_i).wait()

            def cnt_wait(q):
                qg, qk = q // K, lax.rem(q, K)
                pltpu.make_async_copy(
                    ones_v, cnt_s.at[didx.at[lax.rem(qg, 3), qk]],
                    sem_c).wait()

            idx_start(0)
            idx_wait(0)
            idx_start(1)

            def step(j, carry):
                g, k = j // K, lax.rem(j, K)

                @pl.when((k == 0) & (g >= 1))
                def _():
                    @pl.when(g + 1 < G)
                    def _():
                        idx_start(g + 1)
                    idx_wait(g)

                pltpu.async_copy(ones_v,
                                 cnt_s.at[didx.at[lax.rem(g, 3), k]],
                                 sem_c, add=True)

                @pl.when(j >= 2)
                def _():
                    cnt_wait(j - 2)

                return carry

            lax.fori_loop(0, T_SUB, step, 0)
            cnt_wait(T_SUB - 2)
            cnt_wait(T_SUB - 1)

        @pl.when(c == 0)
        def _():
            run(ed_rp, 0)
            run(ed_pr, 1)

        @pl.when(c == 1)
        def _():
            run(ed_rc, 0)
            run(ed_cr, 1)

        plsc.subcore_barrier()

        lo = pl.multiple_of(s * OROWS, 8)
        outs = [(0, 0, o_rp), (1, 0, o_pr), (0, 1, o_rc), (1, 1, o_cr)]
        for slot, cc, out in outs:
            @pl.when((c == cc) & (s < 15))
            def _(slot=slot, out=out):
                pltpu.sync_copy(cnt.at[slot, pl.ds(lo, OROWS)],
                                out.at[pl.ds(lo, OROWS)])

            @pl.when((c == cc) & (s == 15))
            def _(slot=slot, out=out):
                pltpu.sync_copy(cnt.at[slot, pl.ds(15 * OROWS, OLAST)],
                                out.at[pl.ds(15 * OROWS, OLAST)])

    return pl.kernel(body, out_type=out_type, mesh=_mesh(),
                     scratch_types=scratch, compiler_params=_SC_PARAMS)


_count = _make_count()


def _row_spec(w):
    return pl.BlockSpec((BLK, w), lambda i: (i, 0))


def _full_spec(shape):
    return pl.BlockSpec(shape, lambda i: tuple(0 for _ in shape))


def _dot(a, b):
    return lax.dot_general(a, b, (((1,), (0,)), ((), ())),
                           precision=lax.Precision.HIGHEST,
                           preferred_element_type=jnp.float32)


def _dot_t(a, b):
    return lax.dot_general(a, b, (((1,), (1,)), ((), ())),
                           precision=lax.Precision.HIGHEST,
                           preferred_element_type=jnp.float32)


def _proj_body(xp, xc, xr, Wpp, bpp, Wpc, bpc, Wpr, bpr,
               hp0, hp1, hc0, hc1, hr0, hr1):
    p = jnp.maximum(_dot_t(xp[...], Wpp[...]) + bpp[...], 0.0)
    c = jnp.maximum(_dot_t(xc[...], Wpc[...]) + bpc[...], 0.0)
    r = jnp.maximum(_dot_t(xr[...], Wpr[...]) + bpr[...], 0.0)
    hp0[...], hp1[...] = p[:, :32], p[:, 32:]
    hc0[...], hc1[...] = c[:, :32], c[:, 32:]
    hr0[...], hr1[...] = r[:, :32], r[:, 32:]


def _proj(xp, xc, xr, Wpp, bpp, Wpc, bpc, Wpr, bpr):
    h32 = jax.ShapeDtypeStruct((N, 32), jnp.float32)
    return pl.pallas_call(
        _proj_body,
        grid=(N // BLK,),
        in_specs=[_row_spec(5), _row_spec(5), _row_spec(21),
                  _full_spec((64, 5)), _full_spec((1, 64)),
                  _full_spec((64, 5)), _full_spec((1, 64)),
                  _full_spec((64, 21)), _full_spec((1, 64))],
        out_specs=[_row_spec(32)] * 6,
        out_shape=[h32] * 6,
    )(xp, xc, xr, Wpp, bpp, Wpc, bpc, Wpr, bpr)


def _conv2_body(h0, h1, a0, a1, cnt, W, b, o0, o1):
    inv = 1.0 / jnp.clip(cnt[...], 1.0, None)
    cat = jnp.concatenate([a0[...] * inv, a1[...] * inv, h0[...], h1[...]],
                          axis=1)
    y = jnp.maximum(_dot(cat, W[...]) + b[...], 0.0)
    o0[...], o1[...] = y[:, :32], y[:, 32:]


def _conv2(h0, h1, a0, a1, cnt, W, b):
    h32 = jax.ShapeDtypeStruct((N, 32), jnp.float32)
    return pl.pallas_call(
        _conv2_body,
        grid=(N // BLK,),
        in_specs=[_row_spec(32)] * 4 + [_row_spec(1)]
        + [_full_spec((128, 64)), _full_spec((1, 64))],
        out_specs=[_row_spec(32)] * 2,
        out_shape=[h32] * 2,
    )(h0, h1, a0, a1, cnt, W, b)


def _conv3_body(h0, h1, a0, a1, b0, b1, ca, cb, W, bb, o0, o1):
    ia = 1.0 / jnp.clip(ca[...], 1.0, None)
    ib = 1.0 / jnp.clip(cb[...], 1.0, None)
    cat = jnp.concatenate([a0[...] * ia, a1[...] * ia,
                           b0[...] * ib, b1[...] * ib,
                           h0[...], h1[...]], axis=1)
    y = jnp.maximum(_dot(cat, W[...]) + bb[...], 0.0)
    o0[...], o1[...] = y[:, :32], y[:, 32:]


def _conv3(h0, h1, a0, a1, b0, b1, ca, cb, W, bb):
    h32 = jax.ShapeDtypeStruct((N, 32), jnp.float32)
    return pl.pallas_call(
        _conv3_body,
        grid=(N // BLK,),
        in_specs=[_row_spec(32)] * 6 + [_row_spec(1)] * 2
        + [_full_spec((192, 64)), _full_spec((1, 64))],
        out_specs=[_row_spec(32)] * 2,
        out_shape=[h32] * 2,
    )(h0, h1, a0, a1, b0, b1, ca, cb, W, bb)


def _head_body(hr0, hr1, W1, b1, W2, b2, out):
    hcat = jnp.concatenate([hr0[...], hr1[...]], axis=1)
    hid = jnp.maximum(_dot_t(hcat, W1[...]) + b1[...], 0.0)
    out[...] = _dot_t(hid, W2[...]) + b2[0, 0]


def _head(hr0, hr1, Wh1, bh1, Wh2, bh2):
    return pl.pallas_call(
        _head_body,
        grid=(N // BLK,),
        in_specs=[_row_spec(32), _row_spec(32),
                  _full_spec((32, 64)), _full_spec((1, 32)),
                  _full_spec((8, 32)), _full_spec((1, 1))],
        out_specs=_row_spec(8),
        out_shape=jax.ShapeDtypeStruct((N, 8), jnp.float32),
    )(hr0, hr1, Wh1, bh1, Wh2, bh2)


def _pad_edges(e):
    pad = EP - e.shape[1]
    src = jnp.concatenate([e[0], jnp.zeros((pad,), jnp.int32)])
    dst = jnp.concatenate([e[1], jnp.full((pad,), N, jnp.int32)])
    return src.reshape(EROWS, CHUNK), dst.reshape(EROWS, CHUNK)


def _wcat(Wn, Ws):
    return jnp.concatenate([Wn.T, Ws.T], axis=0)


def kernel(x_product, x_customer, x_review, e_rp, e_pr, e_rc, e_cr,
           Wpp, bpp, Wpc, bpc, Wpr, bpr,
           c1_rp_Wn, c1_rp_Ws, c1_rp_bs, c1_pr_Wn, c1_pr_Ws, c1_pr_bs,
           c1_rc_Wn, c1_rc_Ws, c1_rc_bs, c1_cr_Wn, c1_cr_Ws, c1_cr_bs,
           c2_rp_Wn, c2_rp_Ws, c2_rp_bs, c2_pr_Wn, c2_pr_Ws, c2_pr_bs,
           c2_rc_Wn, c2_rc_Ws, c2_rc_bs, c2_cr_Wn, c2_cr_Ws, c2_cr_bs,
           Wh1, bh1, Wh2, bh2):
    r2 = lambda b: b.reshape(1, -1)
    z32 = jnp.zeros((ZROWS, 32), jnp.float32)
    zc = jnp.zeros((ZROWS, CW), jnp.float32)
    ones = jnp.ones((CHUNK, CW), jnp.float32)
    edges = {k: _pad_edges(e) for k, e in
             (("rp", e_rp), ("pr", e_pr), ("rc", e_rc), ("cr", e_cr))}

    c_rp, c_pr, c_rc, c_cr = _count(
        edges["rp"][1], edges["pr"][1], edges["rc"][1], edges["cr"][1],
        zc, ones)
    cnts = {"rp": c_rp[:, :1], "pr": c_pr[:, :1],
            "rc": c_rc[:, :1], "cr": c_cr[:, :1]}

    hp0, hp1, hc0, hc1, hr0, hr1 = _proj(
        x_product, x_customer, x_review,
        Wpp, r2(bpp), Wpc, r2(bpc), Wpr, r2(bpr))

    apr = _agg(hp0, hp1, *edges["pr"], z32)
    acr = _agg(hc0, hc1, *edges["cr"], z32)
    arp = _agg(hr0, hr1, *edges["rp"], z32)
    arc = _agg(hr0, hr1, *edges["rc"], z32)
    Wr1 = jnp.concatenate([c1_pr_Wn.T, c1_cr_Wn.T,
                           c1_pr_Ws.T + c1_cr_Ws.T], axis=0)
    nr0, nr1 = _conv3(hr0, hr1, *apr, *acr, cnts["pr"], cnts["cr"],
                      Wr1, r2(c1_pr_bs + c1_cr_bs))
    np0, np1 = _conv2(hp0, hp1, *arp, cnts["rp"],
                      _wcat(c1_rp_Wn, c1_rp_Ws), r2(c1_rp_bs))
    nc0, nc1 = _conv2(hc0, hc1, *arc, cnts["rc"],
                      _wcat(c1_rc_Wn, c1_rc_Ws), r2(c1_rc_bs))

    apr2 = _agg(np0, np1, *edges["pr"], z32)
    acr2 = _agg(nc0, nc1, *edges["cr"], z32)
    Wr2 = jnp.concatenate([c2_pr_Wn.T, c2_cr_Wn.T,
                           c2_pr_Ws.T + c2_cr_Ws.T], axis=0)
    hr0, hr1 = _conv3(nr0, nr1, *apr2, *acr2, cnts["pr"], cnts["cr"],
                      Wr2, r2(c2_pr_bs + c2_cr_bs))

    Wh2p = jnp.concatenate([Wh2, jnp.zeros((7, 32), jnp.float32)], axis=0)
    out = _head(hr0, hr1, Wh1, r2(bh1), Wh2p, bh2.reshape(1, 1))
    return out[:, 0]


# speedup vs baseline: 1.0241x; 1.0241x over previous
import jax
import jax.numpy as jnp
from jax import lax
from jax.experimental import pallas as pl
from jax.experimental.pallas import tpu as pltpu
from jax.experimental.pallas import tpu_sc as plsc

N = 50000
E = 1000000
NSUB = 16
CHUNK = 128
K = 16
G = 31
T_SUB = K * G
R = 5
D = 4
EP = NSUB * CHUNK * T_SUB
EROWS = NSUB * T_SUB
ZROWS = 3128
ACC_ROWS = NSUB * ZROWS
OROWS = 3128
OLAST = N - 15 * OROWS
CW = 8
BLK = 2000

_SC_PARAMS = pltpu.CompilerParams(use_tc_tiling_on_sc=False)


def _mesh():
    return plsc.VectorSubcoreMesh(core_axis_name="c", subcore_axis_name="s")


def _make_agg():
    out_type = [jax.ShapeDtypeStruct((N, 32), jnp.float32),
                jax.ShapeDtypeStruct((N, 32), jnp.float32)]
    scratch = [
        pltpu.VMEM_SHARED((ACC_ROWS, 32), jnp.float32),
        pltpu.VMEM((2, K, CHUNK), jnp.int32),
        pltpu.VMEM((3, K, CHUNK), jnp.int32),
        pltpu.VMEM((R, CHUNK, 32), jnp.float32),
        pltpu.SemaphoreType.DMA,
        pltpu.SemaphoreType.DMA,
        pltpu.SemaphoreType.DMA,
    ]

    def body(h0, h1, esrc, edst, z32, agg0, agg1, acc, sidx, didx, rows,
             sem_i, sem_g, sem_s):
        c = lax.axis_index("c")
        s = lax.axis_index("s")

        pltpu.sync_copy(z32, acc.at[pl.ds(s * ZROWS, ZROWS)])
        plsc.subcore_barrier()

        B = s * T_SUB

        def idx_start(g):
            pltpu.async_copy(esrc.at[pl.ds(B + g * K, K)],
                             sidx.at[lax.rem(g, 2)], sem_i)
            pltpu.async_copy(edst.at[pl.ds(B + g * K, K)],
                             didx.at[lax.rem(g, 3)], sem_i)

        def idx_wait(g):
            pltpu.make_async_copy(esrc.at[pl.ds(B + g * K, K)],
                                  sidx.at[lax.rem(g, 2)], sem_i).wait()
            pltpu.make_async_copy(edst.at[pl.ds(B + g * K, K)],
                                  didx.at[lax.rem(g, 3)], sem_i).wait()

        def run(h):
            def gather_start(q):
                qg, qk = q // K, lax.rem(q, K)
                pltpu.async_copy(h.at[sidx.at[lax.rem(qg, 2), qk]],
                                 rows.at[lax.rem(q, R)], sem_g)

            def gather_wait(q):
                qg, qk = q // K, lax.rem(q, K)
                pltpu.make_async_copy(h.at[sidx.at[lax.rem(qg, 2), qk]],
                                      rows.at[lax.rem(q, R)], sem_g).wait()

            def scatter_wait(q):
                qg, qk = q // K, lax.rem(q, K)
                pltpu.make_async_copy(
                    rows.at[lax.rem(q, R)],
                    acc.at[didx.at[lax.rem(qg, 3), qk]], sem_s).wait()

            idx_start(0)
            idx_wait(0)
            idx_start(1)
            for q in range(D):
                gather_start(q)

            def step(j, carry):
                g, k = j // K, lax.rem(j, K)
                gather_wait(j)

                @pl.when((k == 0) & (g >= 1) & (g + 1 < G))
                def _():
                    idx_start(g + 1)

                pltpu.async_copy(rows.at[lax.rem(j, R)],
                                 acc.at[didx.at[lax.rem(g, 3), k]],
                                 sem_s, add=True)

                q = j + D

                @pl.when(q < T_SUB)
                def _():
                    @pl.when((lax.rem(q, K) == 0) & (q // K >= 1))
                    def _():
                        idx_wait(q // K)

                    @pl.when(q >= R)
                    def _():
                        scatter_wait(q - R)

                    gather_start(q)

                return carry

            lax.fori_loop(0, T_SUB, step, 0)
            for r in range(R):
                scatter_wait(T_SUB - R + r)

        @pl.when(c == 0)
        def _():
            run(h0)

        @pl.when(c == 1)
        def _():
            run(h1)

        plsc.subcore_barrier()

        lo = pl.multiple_of(s * OROWS, 8)
        agg = [agg0, agg1]
        for cc in (0, 1):
            @pl.when((c == cc) & (s < 15))
            def _(cc=cc):
                pltpu.sync_copy(acc.at[pl.ds(lo, OROWS)],
                                agg[cc].at[pl.ds(lo, OROWS)])

            @pl.when((c == cc) & (s == 15))
            def _(cc=cc):
                pltpu.sync_copy(acc.at[pl.ds(15 * OROWS, OLAST)],
                                agg[cc].at[pl.ds(15 * OROWS, OLAST)])

    return pl.kernel(body, out_type=out_type, mesh=_mesh(),
                     scratch_types=scratch, compiler_params=_SC_PARAMS)


_agg = _make_agg()


def _make_count():
    out_type = [jax.ShapeDtypeStruct((N, CW), jnp.float32)] * 4
    scratch = [
        pltpu.VMEM_SHARED((2, ACC_ROWS, CW), jnp.float32),
        pltpu.VMEM((3, K, CHUNK), jnp.int32),
        pltpu.VMEM((CHUNK, CW), jnp.float32),
        pltpu.SemaphoreType.DMA,
        pltpu.SemaphoreType.DMA,
    ]

    def body(ed_rp, ed_pr, ed_rc, ed_cr, zc, ones,
             o_rp, o_pr, o_rc, o_cr, cnt, didx, ones_v, sem_i, sem_c):
        c = lax.axis_index("c")
        s = lax.axis_index("s")

        pltpu.sync_copy(zc, cnt.at[0, pl.ds(s * ZROWS, ZROWS)])
        pltpu.sync_copy(zc, cnt.at[1, pl.ds(s * ZROWS, ZROWS)])
        pltpu.sync_copy(ones, ones_v)
        plsc.subcore_barrier()

        B = s * T_SUB

        def run(ed, slot):
            cnt_s = cnt.at[slot]

            def idx_start(g):
                pltpu.async_copy(ed.at[pl.ds(B + g * K, K)],
                                 didx.at[lax.rem(g, 3)], sem_i)

            def idx_wait(g):
                pltpu.make_async_copy(ed.at[pl.ds(B + g * K, K)],
                                      didx.at[lax.rem(g, 3)], sem_i).wait()

            def cnt_wait(q):
                qg, qk = q // K, lax.rem(q, K)
                pltpu.make_async_copy(
                    ones_v, cnt_s.at[didx.at[lax.rem(qg, 3), qk]],
                    sem_c).wait()

            idx_start(0)
            idx_wait(0)
            idx_start(1)

            def step(j, carry):
                g, k = j // K, lax.rem(j, K)

                @pl.when((k == 0) & (g >= 1))
                def _():
                    @pl.when(g + 1 < G)
                    def _():
                        idx_start(g + 1)
                    idx_wait(g)

                pltpu.async_copy(ones_v,
                                 cnt_s.at[didx.at[lax.rem(g, 3), k]],
                                 sem_c, add=True)

                @pl.when(j >= 2)
                def _():
                    cnt_wait(j - 2)

                return carry

            lax.fori_loop(0, T_SUB, step, 0)
            cnt_wait(T_SUB - 2)
            cnt_wait(T_SUB - 1)

        @pl.when(c == 0)
        def _():
            run(ed_rp, 0)
            run(ed_pr, 1)

        @pl.when(c == 1)
        def _():
            run(ed_rc, 0)
            run(ed_cr, 1)

        plsc.subcore_barrier()

        lo = pl.multiple_of(s * OROWS, 8)
        outs = [(0, 0, o_rp), (1, 0, o_pr), (0, 1, o_rc), (1, 1, o_cr)]
        for slot, cc, out in outs:
            @pl.when((c == cc) & (s < 15))
            def _(slot=slot, out=out):
                pltpu.sync_copy(cnt.at[slot, pl.ds(lo, OROWS)],
                                out.at[pl.ds(lo, OROWS)])

            @pl.when((c == cc) & (s == 15))
            def _(slot=slot, out=out):
                pltpu.sync_copy(cnt.at[slot, pl.ds(15 * OROWS, OLAST)],
                                out.at[pl.ds(15 * OROWS, OLAST)])

    return pl.kernel(body, out_type=out_type, mesh=_mesh(),
                     scratch_types=scratch, compiler_params=_SC_PARAMS)


_count = _make_count()


def _row_spec(w):
    return pl.BlockSpec((BLK, w), lambda i: (i, 0))


def _full_spec(shape):
    return pl.BlockSpec(shape, lambda i: tuple(0 for _ in shape))


def _dot(a, b):
    return lax.dot_general(a.astype(jnp.bfloat16), b.astype(jnp.bfloat16),
                           (((1,), (0,)), ((), ())),
                           preferred_element_type=jnp.float32)


def _dot_t(a, b):
    return lax.dot_general(a.astype(jnp.bfloat16), b.astype(jnp.bfloat16),
                           (((1,), (1,)), ((), ())),
                           preferred_element_type=jnp.float32)


def _proj_body(xp, xc, xr, Wpp, bpp, Wpc, bpc, Wpr, bpr,
               hp0, hp1, hc0, hc1, hr0, hr1):
    p = jnp.maximum(_dot_t(xp[...], Wpp[...]) + bpp[...], 0.0)
    c = jnp.maximum(_dot_t(xc[...], Wpc[...]) + bpc[...], 0.0)
    r = jnp.maximum(_dot_t(xr[...], Wpr[...]) + bpr[...], 0.0)
    hp0[...], hp1[...] = p[:, :32], p[:, 32:]
    hc0[...], hc1[...] = c[:, :32], c[:, 32:]
    hr0[...], hr1[...] = r[:, :32], r[:, 32:]


def _proj(xp, xc, xr, Wpp, bpp, Wpc, bpc, Wpr, bpr):
    h32 = jax.ShapeDtypeStruct((N, 32), jnp.float32)
    return pl.pallas_call(
        _proj_body,
        grid=(N // BLK,),
        in_specs=[_row_spec(5), _row_spec(5), _row_spec(21),
                  _full_spec((64, 5)), _full_spec((1, 64)),
                  _full_spec((64, 5)), _full_spec((1, 64)),
                  _full_spec((64, 21)), _full_spec((1, 64))],
        out_specs=[_row_spec(32)] * 6,
        out_shape=[h32] * 6,
    )(xp, xc, xr, Wpp, bpp, Wpc, bpc, Wpr, bpr)


def _conv2_body(h0, h1, a0, a1, cnt, W, b, o0, o1):
    inv = 1.0 / jnp.clip(cnt[...], 1.0, None)
    cat = jnp.concatenate([a0[...] * inv, a1[...] * inv, h0[...], h1[...]],
                          axis=1)
    y = jnp.maximum(_dot(cat, W[...]) + b[...], 0.0)
    o0[...], o1[...] = y[:, :32], y[:, 32:]


def _conv2(h0, h1, a0, a1, cnt, W, b):
    h32 = jax.ShapeDtypeStruct((N, 32), jnp.float32)
    return pl.pallas_call(
        _conv2_body,
        grid=(N // BLK,),
        in_specs=[_row_spec(32)] * 4 + [_row_spec(1)]
        + [_full_spec((128, 64)), _full_spec((1, 64))],
        out_specs=[_row_spec(32)] * 2,
        out_shape=[h32] * 2,
    )(h0, h1, a0, a1, cnt, W, b)


def _conv3_body(h0, h1, a0, a1, b0, b1, ca, cb, W, bb, o0, o1):
    ia = 1.0 / jnp.clip(ca[...], 1.0, None)
    ib = 1.0 / jnp.clip(cb[...], 1.0, None)
    cat = jnp.concatenate([a0[...] * ia, a1[...] * ia,
                           b0[...] * ib, b1[...] * ib,
                           h0[...], h1[...], h0[...], h1[...]], axis=1)
    y = jnp.maximum(_dot(cat, W[...]) + bb[...], 0.0)
    o0[...], o1[...] = y[:, :32], y[:, 32:]


def _conv3(h0, h1, a0, a1, b0, b1, ca, cb, W, bb):
    h32 = jax.ShapeDtypeStruct((N, 32), jnp.float32)
    return pl.pallas_call(
        _conv3_body,
        grid=(N // BLK,),
        in_specs=[_row_spec(32)] * 6 + [_row_spec(1)] * 2
        + [_full_spec((256, 64)), _full_spec((1, 64))],
        out_specs=[_row_spec(32)] * 2,
        out_shape=[h32] * 2,
    )(h0, h1, a0, a1, b0, b1, ca, cb, W, bb)


def _head_body(hr0, hr1, W1, b1, W2, b2, out):
    hcat = jnp.concatenate([hr0[...], hr1[...]], axis=1)
    hid = jnp.maximum(_dot_t(hcat, W1[...]) + b1[...], 0.0)
    out[...] = _dot_t(hid, W2[...]) + b2[0, 0]


def _head(hr0, hr1, Wh1, bh1, Wh2, bh2):
    return pl.pallas_call(
        _head_body,
        grid=(N // BLK,),
        in_specs=[_row_spec(32), _row_spec(32),
                  _full_spec((32, 64)), _full_spec((1, 32)),
                  _full_spec((8, 32)), _full_spec((1, 1))],
        out_specs=_row_spec(8),
        out_shape=jax.ShapeDtypeStruct((N, 8), jnp.float32),
    )(hr0, hr1, Wh1, bh1, Wh2, bh2)


def _pad_edges(e):
    pad = EP - e.shape[1]
    src = jnp.concatenate([e[0], jnp.zeros((pad,), jnp.int32)])
    dst = jnp.concatenate([e[1], jnp.full((pad,), N, jnp.int32)])
    return src.reshape(EROWS, CHUNK), dst.reshape(EROWS, CHUNK)


def _wcat(Wn, Ws):
    return jnp.concatenate([Wn.T, Ws.T], axis=0)


def kernel(x_product, x_customer, x_review, e_rp, e_pr, e_rc, e_cr,
           Wpp, bpp, Wpc, bpc, Wpr, bpr,
           c1_rp_Wn, c1_rp_Ws, c1_rp_bs, c1_pr_Wn, c1_pr_Ws, c1_pr_bs,
           c1_rc_Wn, c1_rc_Ws, c1_rc_bs, c1_cr_Wn, c1_cr_Ws, c1_cr_bs,
           c2_rp_Wn, c2_rp_Ws, c2_rp_bs, c2_pr_Wn, c2_pr_Ws, c2_pr_bs,
           c2_rc_Wn, c2_rc_Ws, c2_rc_bs, c2_cr_Wn, c2_cr_Ws, c2_cr_bs,
           Wh1, bh1, Wh2, bh2):
    r2 = lambda b: b.reshape(1, -1)
    z32 = jnp.zeros((ZROWS, 32), jnp.float32)
    zc = jnp.zeros((ZROWS, CW), jnp.float32)
    ones = jnp.ones((CHUNK, CW), jnp.float32)
    edges = {k: _pad_edges(e) for k, e in
             (("rp", e_rp), ("pr", e_pr), ("rc", e_rc), ("cr", e_cr))}

    c_rp, c_pr, c_rc, c_cr = _count(
        edges["rp"][1], edges["pr"][1], edges["rc"][1], edges["cr"][1],
        zc, ones)
    cnts = {"rp": c_rp[:, :1], "pr": c_pr[:, :1],
            "rc": c_rc[:, :1], "cr": c_cr[:, :1]}

    hp0, hp1, hc0, hc1, hr0, hr1 = _proj(
        x_product, x_customer, x_review,
        Wpp, r2(bpp), Wpc, r2(bpc), Wpr, r2(bpr))

    apr = _agg(hp0, hp1, *edges["pr"], z32)
    acr = _agg(hc0, hc1, *edges["cr"], z32)
    arp = _agg(hr0, hr1, *edges["rp"], z32)
    arc = _agg(hr0, hr1, *edges["rc"], z32)
    Wr1 = jnp.concatenate([c1_pr_Wn.T, c1_cr_Wn.T,
                           c1_pr_Ws.T, c1_cr_Ws.T], axis=0)
    nr0, nr1 = _conv3(hr0, hr1, *apr, *acr, cnts["pr"], cnts["cr"],
                      Wr1, r2(c1_pr_bs + c1_cr_bs))
    np0, np1 = _conv2(hp0, hp1, *arp, cnts["rp"],
                      _wcat(c1_rp_Wn, c1_rp_Ws), r2(c1_rp_bs))
    nc0, nc1 = _conv2(hc0, hc1, *arc, cnts["rc"],
                      _wcat(c1_rc_Wn, c1_rc_Ws), r2(c1_rc_bs))

    apr2 = _agg(np0, np1, *edges["pr"], z32)
    acr2 = _agg(nc0, nc1, *edges["cr"], z32)
    Wr2 = jnp.concatenate([c2_pr_Wn.T, c2_cr_Wn.T,
                           c2_pr_Ws.T, c2_cr_Ws.T], axis=0)
    hr0, hr1 = _conv3(nr0, nr1, *apr2, *acr2, cnts["pr"], cnts["cr"],
                      Wr2, r2(c2_pr_bs + c2_cr_bs))

    Wh2p = jnp.concatenate([Wh2, jnp.zeros((7, 32), jnp.float32)], axis=0)
    out = _head(hr0, hr1, Wh1, r2(bh1), Wh2p, bh2.reshape(1, 1))
    return out[:, 0]
